# baseline (speedup 1.0000x reference)
import jax
import jax.numpy as jnp
from jax import lax
from jax.experimental import pallas as pl
from jax.experimental.pallas import tpu as pltpu
from jax.experimental.pallas import tpu_sc as plsc

N_NODES = 10000
D_IN = 128
N_EDGES = 320000
CHUNK = 80
N_CHUNKS = N_EDGES // CHUNK
NW = 32
CPW = N_CHUNKS // NW
IDXB = 25
GSPLIT = 5
ROW_CHUNKS = N_NODES // CHUNK
CROWS = N_NODES // 16


def _sc_body(x_hbm, src_hbm, dst_hbm, z_hbm, sum_hbm, cnt_hbm,
             acc, src_v, dst_v, rows_a, rows_b, cnt_v, sems_a, sems_b):
    cid = lax.axis_index("c")
    sid = lax.axis_index("s")
    wid = sid * 2 + cid

    pltpu.sync_copy(z_hbm, rows_a)

    def zbody(i, carry):
        ck = sid + i * 16

        @pl.when(ck < ROW_CHUNKS)
        def _():
            pltpu.sync_copy(rows_a, acc.at[pl.ds(ck * CHUNK, CHUNK)])

        return carry

    lax.fori_loop(0, (ROW_CHUNKS + 15) // 16, zbody, 0)

    zeros16 = jnp.zeros((16,), jnp.float32)

    def czero(i, carry):
        cnt_v[pl.ds(i * 16, 16)] = zeros16
        return carry

    lax.fori_loop(0, CROWS, czero, 0)
    plsc.subcore_barrier()

    ones16 = jnp.ones((16,), jnp.float32)

    def hist(j):
        def hbody(k, c2):
            idx = dst_v[j, pl.ds(k * 16, 16)]
            plsc.addupdate_scatter(cnt_v, [idx], ones16)
            return c2

        lax.fori_loop(0, CHUNK // 16, hbody, 0)

    H = CHUNK // GSPLIT

    def gather(j, rows, sems):
        for g in range(GSPLIT):
            pltpu.async_copy(x_hbm.at[src_v.at[j, pl.ds(g * H, H)]],
                             rows.at[pl.ds(g * H, H)], sems[g])

    def gwait(j, rows, sems):
        for g in range(GSPLIT):
            pltpu.make_async_copy(x_hbm.at[src_v.at[j, pl.ds(g * H, H)]],
                                  rows.at[pl.ds(g * H, H)], sems[g]).wait()

    def stage(st, carry):
        pltpu.sync_copy(src_hbm.at[wid, st], src_v)
        pltpu.sync_copy(dst_hbm.at[wid, st], dst_v)
        gather(0, rows_a, sems_a)

        def pair(t, c1):
            j0 = 2 * t
            j1 = j0 + 1
            gwait(j0, rows_a, sems_a)
            gather(j1, rows_b, sems_b)
            pltpu.sync_copy(rows_a, acc.at[dst_v.at[j0]], add=True)
            hist(j0)
            gwait(j1, rows_b, sems_b)

            @pl.when(j1 + 1 < IDXB)
            def _():
                gather(j1 + 1, rows_a, sems_a)

            pltpu.sync_copy(rows_b, acc.at[dst_v.at[j1]], add=True)
            hist(j1)
            return c1

        lax.fori_loop(0, IDXB // 2, pair, 0)
        gwait(IDXB - 1, rows_a, sems_a)
        pltpu.sync_copy(rows_a, acc.at[dst_v.at[IDXB - 1]], add=True)
        hist(IDXB - 1)
        return carry

    lax.fori_loop(0, CPW // IDXB, stage, 0)
    plsc.subcore_barrier()

    def wbody(i, carry):
        ck = sid + i * 16

        @pl.when(ck < ROW_CHUNKS)
        def _():
            pltpu.sync_copy(
                acc.at[pl.ds(ck * CHUNK, CHUNK)],
                sum_hbm.at[pl.ds(cid * N_NODES + ck * CHUNK, CHUNK)],
            )

        return carry

    lax.fori_loop(0, (ROW_CHUNKS + 15) // 16, wbody, 0)
    pltpu.sync_copy(cnt_v, cnt_hbm.at[pl.ds(wid * N_NODES, N_NODES)])


_sc_aggregate = pl.kernel(
    _sc_body,
    out_type=(
        jax.ShapeDtypeStruct((2 * N_NODES, D_IN), jnp.float32),
        jax.ShapeDtypeStruct((NW * N_NODES,), jnp.float32),
    ),
    mesh=plsc.VectorSubcoreMesh(core_axis_name="c", subcore_axis_name="s"),
    compiler_params=pltpu.CompilerParams(needs_layout_passes=False),
    scratch_types=[
        pltpu.VMEM_SHARED((N_NODES, D_IN), jnp.float32),
        pltpu.VMEM((IDXB, CHUNK), jnp.int32),
        pltpu.VMEM((IDXB, CHUNK), jnp.int32),
        pltpu.VMEM((CHUNK, D_IN), jnp.float32),
        pltpu.VMEM((CHUNK, D_IN), jnp.float32),
        pltpu.VMEM((N_NODES,), jnp.float32),
        [pltpu.SemaphoreType.DMA] * GSPLIT,
        [pltpu.SemaphoreType.DMA] * GSPLIT,
    ],
)


def _tc_body(p_ref, c_ref, wg_ref, bg_ref, wf_ref, bf_ref, o_ref):
    p = p_ref[...]
    s = p[0] + p[1]
    cnt = jnp.maximum(jnp.sum(c_ref[...], axis=1), 1.0)[:, None]
    aggr = s / cnt
    h = jnp.dot(aggr, wg_ref[...], preferred_element_type=jnp.float32)
    h = h + bg_ref[...]
    o = jnp.dot(h, wf_ref[...], preferred_element_type=jnp.float32)
    o_ref[...] = o + bf_ref[...]


def _tc_dense(partials, counts, wgT, bg, wfT, bf):
    R = 1000
    grid = (N_NODES // R,)
    return pl.pallas_call(
        _tc_body,
        grid=grid,
        in_specs=[
            pl.BlockSpec((2, R, D_IN), lambda i: (0, i, 0)),
            pl.BlockSpec((R, NW), lambda i: (i, 0)),
            pl.BlockSpec((D_IN, D_IN), lambda i: (0, 0)),
            pl.BlockSpec((1, D_IN), lambda i: (0, 0)),
            pl.BlockSpec((D_IN, 8), lambda i: (0, 0)),
            pl.BlockSpec((1, 8), lambda i: (0, 0)),
        ],
        out_specs=pl.BlockSpec((R, 8), lambda i: (i, 0)),
        out_shape=jax.ShapeDtypeStruct((N_NODES, 8), jnp.float32),
    )(partials, counts, wgT, bg, wfT, bf)


@jax.jit
def kernel(x, edge_index, W_gnn, b_gnn, W_fc, b_fc):
    n, d = x.shape
    src = edge_index[0].astype(jnp.int32).reshape(NW, CPW // IDXB, IDXB, CHUNK)
    dst = edge_index[1].astype(jnp.int32).reshape(NW, CPW // IDXB, IDXB, CHUNK)
    zrow = jnp.zeros((CHUNK, D_IN), x.dtype)
    flat, cnt = _sc_aggregate(x, src, dst, zrow)
    partials = flat.reshape(2, N_NODES, D_IN)
    counts = cnt.reshape(NW, N_NODES).T
    wfT = jnp.zeros((D_IN, 8), W_fc.dtype).at[:, :3].set(W_fc.T)
    bf = jnp.zeros((1, 8), b_fc.dtype).at[0, :3].set(b_fc)
    out = _tc_dense(partials, counts, wgT=W_gnn.T,
                    bg=b_gnn.reshape(1, D_IN), wfT=wfT, bf=bf)
    return out[:, :3]


# speedup vs baseline: 1.0095x; 1.0095x over previous
import jax
import jax.numpy as jnp
from jax import lax
from jax.experimental import pallas as pl
from jax.experimental.pallas import tpu as pltpu
from jax.experimental.pallas import tpu_sc as plsc

N_NODES = 10000
D_IN = 128
N_EDGES = 320000
CHUNK = 80
N_CHUNKS = N_EDGES // CHUNK
NW = 32
CPW = N_CHUNKS // NW
IDXB = 25
GSPLIT = 2
ROW_CHUNKS = N_NODES // CHUNK
CROWS = N_NODES // 16


def _sc_body(x_hbm, src_hbm, dst_hbm, z_hbm, sum_hbm, cnt_hbm,
             acc, src_v, dst_v, rows_a, rows_b, cnt_v, sems_a, sems_b):
    cid = lax.axis_index("c")
    sid = lax.axis_index("s")
    wid = sid * 2 + cid

    pltpu.sync_copy(z_hbm, rows_a)

    def zbody(i, carry):
        ck = sid + i * 16

        @pl.when(ck < ROW_CHUNKS)
        def _():
            pltpu.sync_copy(rows_a, acc.at[pl.ds(ck * CHUNK, CHUNK)])

        return carry

    lax.fori_loop(0, (ROW_CHUNKS + 15) // 16, zbody, 0)

    zeros16 = jnp.zeros((16,), jnp.float32)

    def czero(i, carry):
        cnt_v[pl.ds(i * 16, 16)] = zeros16
        return carry

    lax.fori_loop(0, CROWS, czero, 0)
    plsc.subcore_barrier()

    ones16 = jnp.ones((16,), jnp.float32)

    def hist(j):
        def hbody(k, c2):
            idx = dst_v[j, pl.ds(k * 16, 16)]
            plsc.addupdate_scatter(cnt_v, [idx], ones16)
            return c2

        lax.fori_loop(0, CHUNK // 16, hbody, 0)

    H = CHUNK // GSPLIT

    def gather(j, rows, sems):
        for g in range(GSPLIT):
            pltpu.async_copy(x_hbm.at[src_v.at[j, pl.ds(g * H, H)]],
                             rows.at[pl.ds(g * H, H)], sems[g])

    def gwait(j, rows, sems):
        for g in range(GSPLIT):
            pltpu.make_async_copy(x_hbm.at[src_v.at[j, pl.ds(g * H, H)]],
                                  rows.at[pl.ds(g * H, H)], sems[g]).wait()

    def stage(st, carry):
        pltpu.sync_copy(src_hbm.at[wid, st], src_v)
        pltpu.sync_copy(dst_hbm.at[wid, st], dst_v)
        gather(0, rows_a, sems_a)

        def pair(t, c1):
            j0 = 2 * t
            j1 = j0 + 1
            gwait(j0, rows_a, sems_a)
            gather(j1, rows_b, sems_b)
            pltpu.sync_copy(rows_a, acc.at[dst_v.at[j0]], add=True)
            hist(j0)
            gwait(j1, rows_b, sems_b)

            @pl.when(j1 + 1 < IDXB)
            def _():
                gather(j1 + 1, rows_a, sems_a)

            pltpu.sync_copy(rows_b, acc.at[dst_v.at[j1]], add=True)
            hist(j1)
            return c1

        lax.fori_loop(0, IDXB // 2, pair, 0)
        gwait(IDXB - 1, rows_a, sems_a)
        pltpu.sync_copy(rows_a, acc.at[dst_v.at[IDXB - 1]], add=True)
        hist(IDXB - 1)
        return carry

    lax.fori_loop(0, CPW // IDXB, stage, 0)
    plsc.subcore_barrier()

    def wbody(i, carry):
        ck = sid + i * 16

        @pl.when(ck < ROW_CHUNKS)
        def _():
            pltpu.sync_copy(
                acc.at[pl.ds(ck * CHUNK, CHUNK)],
                sum_hbm.at[pl.ds(cid * N_NODES + ck * CHUNK, CHUNK)],
            )

        return carry

    lax.fori_loop(0, (ROW_CHUNKS + 15) // 16, wbody, 0)
    pltpu.sync_copy(cnt_v, cnt_hbm.at[pl.ds(wid * N_NODES, N_NODES)])


_sc_aggregate = pl.kernel(
    _sc_body,
    out_type=(
        jax.ShapeDtypeStruct((2 * N_NODES, D_IN), jnp.float32),
        jax.ShapeDtypeStruct((NW * N_NODES,), jnp.float32),
    ),
    mesh=plsc.VectorSubcoreMesh(core_axis_name="c", subcore_axis_name="s"),
    compiler_params=pltpu.CompilerParams(needs_layout_passes=False),
    scratch_types=[
        pltpu.VMEM_SHARED((N_NODES, D_IN), jnp.float32),
        pltpu.VMEM((IDXB, CHUNK), jnp.int32),
        pltpu.VMEM((IDXB, CHUNK), jnp.int32),
        pltpu.VMEM((CHUNK, D_IN), jnp.float32),
        pltpu.VMEM((CHUNK, D_IN), jnp.float32),
        pltpu.VMEM((N_NODES,), jnp.float32),
        [pltpu.SemaphoreType.DMA] * GSPLIT,
        [pltpu.SemaphoreType.DMA] * GSPLIT,
    ],
)


def _tc_body(p_ref, c_ref, wg_ref, bg_ref, wf_ref, bf_ref, o_ref):
    p = p_ref[...]
    s = p[0] + p[1]
    cnt = jnp.maximum(jnp.sum(c_ref[...], axis=1), 1.0)[:, None]
    aggr = s / cnt
    h = jnp.dot(aggr, wg_ref[...], preferred_element_type=jnp.float32)
    h = h + bg_ref[...]
    o = jnp.dot(h, wf_ref[...], preferred_element_type=jnp.float32)
    o_ref[...] = o + bf_ref[...]


def _tc_dense(partials, counts, wgT, bg, wfT, bf):
    R = 1000
    grid = (N_NODES // R,)
    return pl.pallas_call(
        _tc_body,
        grid=grid,
        in_specs=[
            pl.BlockSpec((2, R, D_IN), lambda i: (0, i, 0)),
            pl.BlockSpec((R, NW), lambda i: (i, 0)),
            pl.BlockSpec((D_IN, D_IN), lambda i: (0, 0)),
            pl.BlockSpec((1, D_IN), lambda i: (0, 0)),
            pl.BlockSpec((D_IN, 8), lambda i: (0, 0)),
            pl.BlockSpec((1, 8), lambda i: (0, 0)),
        ],
        out_specs=pl.BlockSpec((R, 8), lambda i: (i, 0)),
        out_shape=jax.ShapeDtypeStruct((N_NODES, 8), jnp.float32),
    )(partials, counts, wgT, bg, wfT, bf)


@jax.jit
def kernel(x, edge_index, W_gnn, b_gnn, W_fc, b_fc):
    n, d = x.shape
    src = edge_index[0].astype(jnp.int32).reshape(NW, CPW // IDXB, IDXB, CHUNK)
    dst = edge_index[1].astype(jnp.int32).reshape(NW, CPW // IDXB, IDXB, CHUNK)
    zrow = jnp.zeros((CHUNK, D_IN), x.dtype)
    flat, cnt = _sc_aggregate(x, src, dst, zrow)
    partials = flat.reshape(2, N_NODES, D_IN)
    counts = cnt.reshape(NW, N_NODES).T
    wfT = jnp.zeros((D_IN, 8), W_fc.dtype).at[:, :3].set(W_fc.T)
    bf = jnp.zeros((1, 8), b_fc.dtype).at[0, :3].set(b_fc)
    out = _tc_dense(partials, counts, wgT=W_gnn.T,
                    bg=b_gnn.reshape(1, D_IN), wfT=wfT, bf=bf)
    return out[:, :3]


# speedup vs baseline: 1.0671x; 1.0571x over previous
import jax
import jax.numpy as jnp
from jax import lax
from jax.experimental import pallas as pl
from jax.experimental.pallas import tpu as pltpu
from jax.experimental.pallas import tpu_sc as plsc

N_NODES = 10000
D_IN = 128
N_EDGES = 320000
CHUNK = 80
N_CHUNKS = N_EDGES // CHUNK
NW = 32
CPW = N_CHUNKS // NW
IDXB = 25
GSPLIT = 2
ROW_CHUNKS = N_NODES // CHUNK
CROWS = N_NODES // 16


def _sc_body(x_hbm, ei_hbm, z_hbm, sum_hbm, cnt_hbm,
             acc, src_v, dst_v, rows_a, rows_b, cnt_v, sems_a, sems_b):
    cid = lax.axis_index("c")
    sid = lax.axis_index("s")
    wid = sid * 2 + cid

    pltpu.sync_copy(z_hbm, rows_a)

    def zbody(i, carry):
        ck = sid + i * 16

        @pl.when(ck < ROW_CHUNKS)
        def _():
            pltpu.sync_copy(rows_a, acc.at[pl.ds(ck * CHUNK, CHUNK)])

        return carry

    lax.fori_loop(0, (ROW_CHUNKS + 15) // 16, zbody, 0)

    zeros16 = jnp.zeros((16,), jnp.float32)

    def czero(i, carry):
        cnt_v[pl.ds(i * 16, 16)] = zeros16
        return carry

    lax.fori_loop(0, CROWS, czero, 0)
    plsc.subcore_barrier()

    ones16 = jnp.ones((16,), jnp.float32)

    def hist(j):
        def hbody(k, c2):
            idx = dst_v[j, pl.ds(k * 16, 16)]
            plsc.addupdate_scatter(cnt_v, [idx], ones16)
            return c2

        lax.fori_loop(0, CHUNK // 16, hbody, 0)

    H = CHUNK // GSPLIT

    def gather(j, rows, sems):
        for g in range(GSPLIT):
            pltpu.async_copy(x_hbm.at[src_v.at[j, pl.ds(g * H, H)]],
                             rows.at[pl.ds(g * H, H)], sems[g])

    def gwait(j, rows, sems):
        for g in range(GSPLIT):
            pltpu.make_async_copy(x_hbm.at[src_v.at[j, pl.ds(g * H, H)]],
                                  rows.at[pl.ds(g * H, H)], sems[g]).wait()

    def stage(st, carry):
        pltpu.sync_copy(ei_hbm.at[0, wid, st], src_v)
        pltpu.sync_copy(ei_hbm.at[1, wid, st], dst_v)
        gather(0, rows_a, sems_a)

        def pair(t, c1):
            j0 = 2 * t
            j1 = j0 + 1
            gwait(j0, rows_a, sems_a)
            gather(j1, rows_b, sems_b)
            pltpu.sync_copy(rows_a, acc.at[dst_v.at[j0]], add=True)
            hist(j0)
            gwait(j1, rows_b, sems_b)

            @pl.when(j1 + 1 < IDXB)
            def _():
                gather(j1 + 1, rows_a, sems_a)

            pltpu.sync_copy(rows_b, acc.at[dst_v.at[j1]], add=True)
            hist(j1)
            return c1

        lax.fori_loop(0, IDXB // 2, pair, 0)
        gwait(IDXB - 1, rows_a, sems_a)
        pltpu.sync_copy(rows_a, acc.at[dst_v.at[IDXB - 1]], add=True)
        hist(IDXB - 1)
        return carry

    lax.fori_loop(0, CPW // IDXB, stage, 0)
    plsc.subcore_barrier()

    def wbody(i, carry):
        ck = sid + i * 16

        @pl.when(ck < ROW_CHUNKS)
        def _():
            pltpu.sync_copy(
                acc.at[pl.ds(ck * CHUNK, CHUNK)],
                sum_hbm.at[pl.ds(cid * N_NODES + ck * CHUNK, CHUNK)],
            )

        return carry

    lax.fori_loop(0, (ROW_CHUNKS + 15) // 16, wbody, 0)
    pltpu.sync_copy(cnt_v, cnt_hbm.at[pl.ds(wid * N_NODES, N_NODES)])


_sc_aggregate = pl.kernel(
    _sc_body,
    out_type=(
        jax.ShapeDtypeStruct((2 * N_NODES, D_IN), jnp.float32),
        jax.ShapeDtypeStruct((NW * N_NODES,), jnp.float32),
    ),
    mesh=plsc.VectorSubcoreMesh(core_axis_name="c", subcore_axis_name="s"),
    compiler_params=pltpu.CompilerParams(needs_layout_passes=False),
    scratch_types=[
        pltpu.VMEM_SHARED((N_NODES, D_IN), jnp.float32),
        pltpu.VMEM((IDXB, CHUNK), jnp.int32),
        pltpu.VMEM((IDXB, CHUNK), jnp.int32),
        pltpu.VMEM((CHUNK, D_IN), jnp.float32),
        pltpu.VMEM((CHUNK, D_IN), jnp.float32),
        pltpu.VMEM((N_NODES,), jnp.float32),
        [pltpu.SemaphoreType.DMA] * GSPLIT,
        [pltpu.SemaphoreType.DMA] * GSPLIT,
    ],
)


def _tc_body(p_ref, c_ref, wg_ref, bg_ref, wf_ref, bf_ref, o_ref):
    p = p_ref[...]
    s = p[0] + p[1]
    cnt = jnp.maximum(jnp.sum(c_ref[...], axis=1), 1.0)[:, None]
    k = jax.lax.dot_general(wg_ref[...], wf_ref[...],
                            (((0,), (1,)), ((), ())),
                            preferred_element_type=jnp.float32)
    t = jnp.dot(s, k, preferred_element_type=jnp.float32)
    b = jax.lax.dot_general(bg_ref[...], wf_ref[...],
                            (((1,), (1,)), ((), ())),
                            preferred_element_type=jnp.float32)
    o_ref[...] = t / cnt + b + bf_ref[...]


def _tc_dense(partials, counts, wg, bg, wf, bf):
    R = 1000
    grid = (N_NODES // R,)
    return pl.pallas_call(
        _tc_body,
        grid=grid,
        in_specs=[
            pl.BlockSpec((2, R, D_IN), lambda i: (0, i, 0)),
            pl.BlockSpec((R, NW), lambda i: (i, 0)),
            pl.BlockSpec((D_IN, D_IN), lambda i: (0, 0)),
            pl.BlockSpec((1, D_IN), lambda i: (0, 0)),
            pl.BlockSpec((3, D_IN), lambda i: (0, 0)),
            pl.BlockSpec((1, 3), lambda i: (0, 0)),
        ],
        out_specs=pl.BlockSpec((R, 3), lambda i: (i, 0)),
        out_shape=jax.ShapeDtypeStruct((N_NODES, 3), jnp.float32),
    )(partials, counts, wg, bg, wf, bf)


@jax.jit
def kernel(x, edge_index, W_gnn, b_gnn, W_fc, b_fc):
    ei = edge_index.astype(jnp.int32).reshape(
        2, NW, CPW // IDXB, IDXB, CHUNK)
    zrow = jnp.zeros((CHUNK, D_IN), x.dtype)
    flat, cnt = _sc_aggregate(x, ei, zrow)
    partials = flat.reshape(2, N_NODES, D_IN)
    counts = cnt.reshape(NW, N_NODES).T
    return _tc_dense(partials, counts, W_gnn, b_gnn.reshape(1, D_IN),
                     W_fc, b_fc.reshape(1, 3))


# speedup vs baseline: 1.0698x; 1.0025x over previous
import jax
import jax.numpy as jnp
from jax import lax
from jax.experimental import pallas as pl
from jax.experimental.pallas import tpu as pltpu
from jax.experimental.pallas import tpu_sc as plsc

N_NODES = 10000
D_IN = 128
N_EDGES = 320000
CHUNK = 80
N_CHUNKS = N_EDGES // CHUNK
NW = 32
CPW = N_CHUNKS // NW
IDXB = 25
GSPLIT = 2
ROW_CHUNKS = N_NODES // CHUNK
CROWS = N_NODES // 16


def _sc_body(x_hbm, ei_hbm, z_hbm, sum_hbm, cnt_hbm,
             acc, src_v, dst_v, rows_a, rows_b, cnt_v, sems_a, sems_b):
    cid = lax.axis_index("c")
    sid = lax.axis_index("s")
    wid = sid * 2 + cid

    pltpu.sync_copy(z_hbm, rows_a)

    def zbody(i, carry):
        ck = sid + i * 16

        @pl.when(ck < ROW_CHUNKS)
        def _():
            pltpu.sync_copy(rows_a, acc.at[pl.ds(ck * CHUNK, CHUNK)])

        return carry

    lax.fori_loop(0, (ROW_CHUNKS + 15) // 16, zbody, 0)

    zeros16 = jnp.zeros((16,), jnp.float32)

    def czero(i, carry):
        cnt_v[pl.ds(i * 16, 16)] = zeros16
        return carry

    lax.fori_loop(0, CROWS, czero, 0)
    plsc.subcore_barrier()

    ones16 = jnp.ones((16,), jnp.float32)

    def hist(j):
        def hbody(k, c2):
            idx = dst_v[j, pl.ds(k * 16, 16)]
            plsc.addupdate_scatter(cnt_v, [idx], ones16)
            return c2

        lax.fori_loop(0, CHUNK // 16, hbody, 0)

    H = CHUNK // GSPLIT

    def gather(j, rows, sems):
        for g in range(GSPLIT):
            pltpu.async_copy(x_hbm.at[src_v.at[j, pl.ds(g * H, H)]],
                             rows.at[pl.ds(g * H, H)], sems[g])

    def gwait(j, rows, sems):
        for g in range(GSPLIT):
            pltpu.make_async_copy(x_hbm.at[src_v.at[j, pl.ds(g * H, H)]],
                                  rows.at[pl.ds(g * H, H)], sems[g]).wait()

    def stage(st, carry):
        pltpu.sync_copy(ei_hbm.at[0, wid, st], src_v)
        pltpu.sync_copy(ei_hbm.at[1, wid, st], dst_v)
        gather(0, rows_a, sems_a)

        def pair(t, c1):
            j0 = 2 * t
            j1 = j0 + 1
            gwait(j0, rows_a, sems_a)
            gather(j1, rows_b, sems_b)
            pltpu.sync_copy(rows_a, acc.at[dst_v.at[j0]], add=True)
            hist(j0)
            gwait(j1, rows_b, sems_b)

            @pl.when(j1 + 1 < IDXB)
            def _():
                gather(j1 + 1, rows_a, sems_a)

            pltpu.sync_copy(rows_b, acc.at[dst_v.at[j1]], add=True)
            hist(j1)
            return c1

        lax.fori_loop(0, IDXB // 2, pair, 0)
        gwait(IDXB - 1, rows_a, sems_a)
        pltpu.sync_copy(rows_a, acc.at[dst_v.at[IDXB - 1]], add=True)
        hist(IDXB - 1)
        return carry

    lax.fori_loop(0, CPW // IDXB, stage, 0)
    plsc.subcore_barrier()

    def wbody(i, carry):
        ck = sid + i * 16

        @pl.when(ck < ROW_CHUNKS)
        def _():
            pltpu.sync_copy(
                acc.at[pl.ds(ck * CHUNK, CHUNK)],
                sum_hbm.at[pl.ds(cid * N_NODES + ck * CHUNK, CHUNK)],
            )

        return carry

    lax.fori_loop(0, (ROW_CHUNKS + 15) // 16, wbody, 0)
    pltpu.sync_copy(cnt_v, cnt_hbm.at[pl.ds(wid * N_NODES, N_NODES)])


_sc_aggregate = pl.kernel(
    _sc_body,
    out_type=(
        jax.ShapeDtypeStruct((2 * N_NODES, D_IN), jnp.float32),
        jax.ShapeDtypeStruct((NW * N_NODES,), jnp.float32),
    ),
    mesh=plsc.VectorSubcoreMesh(core_axis_name="c", subcore_axis_name="s"),
    compiler_params=pltpu.CompilerParams(needs_layout_passes=False),
    scratch_types=[
        pltpu.VMEM_SHARED((N_NODES, D_IN), jnp.float32),
        pltpu.VMEM((IDXB, CHUNK), jnp.int32),
        pltpu.VMEM((IDXB, CHUNK), jnp.int32),
        pltpu.VMEM((CHUNK, D_IN), jnp.float32),
        pltpu.VMEM((CHUNK, D_IN), jnp.float32),
        pltpu.VMEM((N_NODES,), jnp.float32),
        [pltpu.SemaphoreType.DMA] * GSPLIT,
        [pltpu.SemaphoreType.DMA] * GSPLIT,
    ],
)


def _tc_body(p_ref, c_ref, wg_ref, bg_ref, wf_ref, bf_ref, o_ref):
    p = p_ref[...]
    s = p[0] + p[1]
    cnt = jnp.maximum(jnp.sum(c_ref[...], axis=1), 1.0)[:, None]
    k = jax.lax.dot_general(wg_ref[...], wf_ref[...],
                            (((0,), (1,)), ((), ())),
                            preferred_element_type=jnp.float32)
    t = jnp.dot(s / cnt, k, preferred_element_type=jnp.float32)
    b = jax.lax.dot_general(bg_ref[...], wf_ref[...],
                            (((1,), (1,)), ((), ())),
                            preferred_element_type=jnp.float32)
    o_ref[...] = t + b + bf_ref[...]


def _tc_dense(partials, counts, wg, bg, wf, bf):
    R = 1000
    grid = (N_NODES // R,)
    return pl.pallas_call(
        _tc_body,
        grid=grid,
        in_specs=[
            pl.BlockSpec((2, R, D_IN), lambda i: (0, i, 0)),
            pl.BlockSpec((R, NW), lambda i: (i, 0)),
            pl.BlockSpec((D_IN, D_IN), lambda i: (0, 0)),
            pl.BlockSpec((1, D_IN), lambda i: (0, 0)),
            pl.BlockSpec((3, D_IN), lambda i: (0, 0)),
            pl.BlockSpec((1, 3), lambda i: (0, 0)),
        ],
        out_specs=pl.BlockSpec((R, 3), lambda i: (i, 0)),
        out_shape=jax.ShapeDtypeStruct((N_NODES, 3), jnp.float32),
    )(partials, counts, wg, bg, wf, bf)


@jax.jit
def kernel(x, edge_index, W_gnn, b_gnn, W_fc, b_fc):
    ei = edge_index.astype(jnp.int32).reshape(
        2, NW, CPW // IDXB, IDXB, CHUNK)
    zrow = jnp.zeros((CHUNK, D_IN), x.dtype)
    flat, cnt = _sc_aggregate(x, ei, zrow)
    partials = flat.reshape(2, N_NODES, D_IN)
    counts = cnt.reshape(NW, N_NODES).T
    return _tc_dense(partials, counts, W_gnn, b_gnn.reshape(1, D_IN),
                     W_fc, b_fc.reshape(1, 3))


# speedup vs baseline: 1.1196x; 1.0465x over previous
import jax
import jax.numpy as jnp
from jax import lax
from jax.experimental import pallas as pl
from jax.experimental.pallas import tpu as pltpu
from jax.experimental.pallas import tpu_sc as plsc

N_NODES = 10000
D_IN = 128
N_EDGES = 320000
CHUNK = 80
N_CHUNKS = N_EDGES // CHUNK
NW = 32
CPW = N_CHUNKS // NW
IDXB = 25
GSPLIT = 2
ROW_CHUNKS = N_NODES // CHUNK
CROWS = N_NODES // 16


def _sc_body(x_hbm, ei_hbm, z_hbm, sum_hbm, cnt_hbm,
             acc, src_v, dst_v, rows_a, rows_b, cnt_v, sems_a, sems_b,
             isem, zsem, wsem):
    cid = lax.axis_index("c")
    sid = lax.axis_index("s")
    wid = sid * 2 + cid

    def idx_load(st, slot):
        pltpu.async_copy(ei_hbm.at[0, wid, st],
                         src_v.at[slot, pl.ds(0, IDXB)], isem)
        pltpu.async_copy(ei_hbm.at[1, wid, st],
                         dst_v.at[slot, pl.ds(0, IDXB)], isem)

    def idx_wait(st, slot):
        pltpu.make_async_copy(ei_hbm.at[0, wid, st],
                              src_v.at[slot, pl.ds(0, IDXB)], isem).wait()
        pltpu.make_async_copy(ei_hbm.at[1, wid, st],
                              dst_v.at[slot, pl.ds(0, IDXB)], isem).wait()

    pltpu.sync_copy(z_hbm, rows_a)
    idx_load(0, 0)

    def zfire(i, carry):
        ck = sid + i * 16

        @pl.when(ck < ROW_CHUNKS)
        def _():
            pltpu.async_copy(rows_a, acc.at[pl.ds(ck * CHUNK, CHUNK)], zsem)

        return carry

    lax.fori_loop(0, (ROW_CHUNKS + 15) // 16, zfire, 0)

    zeros16 = jnp.zeros((16,), jnp.float32)

    def czero(i, carry):
        cnt_v[pl.ds(i * 16, 16)] = zeros16
        return carry

    lax.fori_loop(0, CROWS, czero, 0)

    def zdrain(i, carry):
        ck = sid + i * 16

        @pl.when(ck < ROW_CHUNKS)
        def _():
            pltpu.make_async_copy(rows_a, acc.at[pl.ds(ck * CHUNK, CHUNK)],
                                  zsem).wait()

        return carry

    lax.fori_loop(0, (ROW_CHUNKS + 15) // 16, zdrain, 0)
    plsc.subcore_barrier()

    ones16 = jnp.ones((16,), jnp.float32)

    def hist(p, j):
        def hbody(k, c2):
            idx = dst_v[p, j, pl.ds(k * 16, 16)]
            plsc.addupdate_scatter(cnt_v, [idx], ones16)
            return c2

        lax.fori_loop(0, CHUNK // 16, hbody, 0)

    H = CHUNK // GSPLIT

    def gather(p, j, rows, sems):
        for g in range(GSPLIT):
            pltpu.async_copy(x_hbm.at[src_v.at[p, j, pl.ds(g * H, H)]],
                             rows.at[pl.ds(g * H, H)], sems[g])

    def gwait(p, j, rows, sems):
        for g in range(GSPLIT):
            pltpu.make_async_copy(x_hbm.at[src_v.at[p, j, pl.ds(g * H, H)]],
                                  rows.at[pl.ds(g * H, H)], sems[g]).wait()

    def stage(st, p):
        idx_wait(st, p)

        @pl.when(st + 1 < CPW // IDXB)
        def _():
            idx_load(st + 1, 1 - p)

        gather(p, 0, rows_a, sems_a)

        def pair(t, c1):
            j0 = 2 * t
            j1 = j0 + 1
            gwait(p, j0, rows_a, sems_a)
            gather(p, j1, rows_b, sems_b)
            pltpu.sync_copy(rows_a, acc.at[dst_v.at[p, j0]], add=True)
            hist(p, j0)
            gwait(p, j1, rows_b, sems_b)

            @pl.when(j1 + 1 < IDXB)
            def _():
                gather(p, j1 + 1, rows_a, sems_a)

            pltpu.sync_copy(rows_b, acc.at[dst_v.at[p, j1]], add=True)
            hist(p, j1)
            return c1

        lax.fori_loop(0, IDXB // 2, pair, 0)
        gwait(p, IDXB - 1, rows_a, sems_a)
        pltpu.sync_copy(rows_a, acc.at[dst_v.at[p, IDXB - 1]], add=True)
        hist(p, IDXB - 1)

    def stagepair(q, carry):
        stage(2 * q, 0)
        stage(2 * q + 1, 1)
        return carry

    n_stages = CPW // IDXB
    lax.fori_loop(0, n_stages // 2, stagepair, 0)
    if n_stages % 2:
        stage(n_stages - 1, 0)
    plsc.subcore_barrier()

    def wfire(i, carry):
        ck = sid + i * 16

        @pl.when(ck < ROW_CHUNKS)
        def _():
            pltpu.async_copy(
                acc.at[pl.ds(ck * CHUNK, CHUNK)],
                sum_hbm.at[pl.ds(cid * N_NODES + ck * CHUNK, CHUNK)],
                wsem,
            )

        return carry

    lax.fori_loop(0, (ROW_CHUNKS + 15) // 16, wfire, 0)
    pltpu.sync_copy(cnt_v, cnt_hbm.at[pl.ds(wid * N_NODES, N_NODES)])

    def wdrain(i, carry):
        ck = sid + i * 16

        @pl.when(ck < ROW_CHUNKS)
        def _():
            pltpu.make_async_copy(
                acc.at[pl.ds(ck * CHUNK, CHUNK)],
                sum_hbm.at[pl.ds(cid * N_NODES + ck * CHUNK, CHUNK)],
                wsem,
            ).wait()

        return carry

    lax.fori_loop(0, (ROW_CHUNKS + 15) // 16, wdrain, 0)


_sc_aggregate = pl.kernel(
    _sc_body,
    out_type=(
        jax.ShapeDtypeStruct((2 * N_NODES, D_IN), jnp.float32),
        jax.ShapeDtypeStruct((NW * N_NODES,), jnp.float32),
    ),
    mesh=plsc.VectorSubcoreMesh(core_axis_name="c", subcore_axis_name="s"),
    compiler_params=pltpu.CompilerParams(needs_layout_passes=False),
    scratch_types=[
        pltpu.VMEM_SHARED((N_NODES, D_IN), jnp.float32),
        pltpu.VMEM((2, 32, CHUNK), jnp.int32),
        pltpu.VMEM((2, 32, CHUNK), jnp.int32),
        pltpu.VMEM((CHUNK, D_IN), jnp.float32),
        pltpu.VMEM((CHUNK, D_IN), jnp.float32),
        pltpu.VMEM((N_NODES,), jnp.float32),
        [pltpu.SemaphoreType.DMA] * GSPLIT,
        [pltpu.SemaphoreType.DMA] * GSPLIT,
        pltpu.SemaphoreType.DMA,
        pltpu.SemaphoreType.DMA,
        pltpu.SemaphoreType.DMA,
    ],
)


def _tc_body(p_ref, c_ref, wg_ref, bg_ref, wf_ref, bf_ref, o_ref):
    p = p_ref[...]
    s = p[0] + p[1]
    cnt = jnp.maximum(jnp.sum(c_ref[...], axis=1), 1.0)[:, None]
    k = jax.lax.dot_general(wg_ref[...], wf_ref[...],
                            (((0,), (1,)), ((), ())),
                            preferred_element_type=jnp.float32)
    t = jnp.dot(s / cnt, k, preferred_element_type=jnp.float32)
    b = jax.lax.dot_general(bg_ref[...], wf_ref[...],
                            (((1,), (1,)), ((), ())),
                            preferred_element_type=jnp.float32)
    o_ref[...] = t + b + bf_ref[...]


def _tc_dense(partials, counts, wg, bg, wf, bf):
    R = 1000
    grid = (N_NODES // R,)
    return pl.pallas_call(
        _tc_body,
        grid=grid,
        in_specs=[
            pl.BlockSpec((2, R, D_IN), lambda i: (0, i, 0)),
            pl.BlockSpec((R, NW), lambda i: (i, 0)),
            pl.BlockSpec((D_IN, D_IN), lambda i: (0, 0)),
            pl.BlockSpec((1, D_IN), lambda i: (0, 0)),
            pl.BlockSpec((3, D_IN), lambda i: (0, 0)),
            pl.BlockSpec((1, 3), lambda i: (0, 0)),
        ],
        out_specs=pl.BlockSpec((R, 3), lambda i: (i, 0)),
        out_shape=jax.ShapeDtypeStruct((N_NODES, 3), jnp.float32),
    )(partials, counts, wg, bg, wf, bf)


@jax.jit
def kernel(x, edge_index, W_gnn, b_gnn, W_fc, b_fc):
    ei = edge_index.astype(jnp.int32).reshape(
        2, NW, CPW // IDXB, IDXB, CHUNK)
    zrow = jnp.zeros((CHUNK, D_IN), x.dtype)
    flat, cnt = _sc_aggregate(x, ei, zrow)
    partials = flat.reshape(2, N_NODES, D_IN)
    counts = cnt.reshape(NW, N_NODES).T
    return _tc_dense(partials, counts, W_gnn, b_gnn.reshape(1, D_IN),
                     W_fc, b_fc.reshape(1, 3))


# speedup vs baseline: 1.4290x; 1.2764x over previous
import jax
import jax.numpy as jnp
from jax import lax
from jax.experimental import pallas as pl
from jax.experimental.pallas import tpu as pltpu
from jax.experimental.pallas import tpu_sc as plsc

N_NODES = 10000
D_IN = 128
N_EDGES = 320000
CHUNK = 80
N_CHUNKS = N_EDGES // CHUNK
NW = 32
CPW = N_CHUNKS // NW
IDXB = 25
GSPLIT = 2
ROW_CHUNKS = N_NODES // CHUNK
CROWS = N_NODES // 16


def _sc_body(x_hbm, ei_hbm, z_hbm, sum_hbm, cnt_hbm,
             acc, src_v, dst_v, rows_a, rows_b, rows_c, cnt_v,
             sems_a, sems_b, sems_c, isem, zsem, wsem):
    cid = lax.axis_index("c")
    sid = lax.axis_index("s")
    wid = sid * 2 + cid

    def idx_load(st):
        pltpu.async_copy(ei_hbm.at[0, wid, st], src_v, isem)
        pltpu.async_copy(ei_hbm.at[1, wid, st], dst_v, isem)

    def idx_wait(st):
        pltpu.make_async_copy(ei_hbm.at[0, wid, st], src_v, isem).wait()
        pltpu.make_async_copy(ei_hbm.at[1, wid, st], dst_v, isem).wait()

    pltpu.sync_copy(z_hbm, rows_a)
    idx_load(0)

    def zfire(i, carry):
        ck = sid + i * 16

        @pl.when(ck < ROW_CHUNKS)
        def _():
            pltpu.async_copy(rows_a, acc.at[pl.ds(ck * CHUNK, CHUNK)], zsem)

        return carry

    lax.fori_loop(0, (ROW_CHUNKS + 15) // 16, zfire, 0)

    zeros16 = jnp.zeros((16,), jnp.float32)

    def czero(i, carry):
        cnt_v[pl.ds(i * 16, 16)] = zeros16
        return carry

    lax.fori_loop(0, CROWS, czero, 0)

    def zdrain(i, carry):
        ck = sid + i * 16

        @pl.when(ck < ROW_CHUNKS)
        def _():
            pltpu.make_async_copy(rows_a, acc.at[pl.ds(ck * CHUNK, CHUNK)],
                                  zsem).wait()

        return carry

    lax.fori_loop(0, (ROW_CHUNKS + 15) // 16, zdrain, 0)
    plsc.subcore_barrier()

    ones16 = jnp.ones((16,), jnp.float32)

    def hist(j):
        def hbody(k, c2):
            idx = dst_v[j, pl.ds(k * 16, 16)]
            plsc.addupdate_scatter(cnt_v, [idx], ones16)
            return c2

        lax.fori_loop(0, CHUNK // 16, hbody, 0)

    H = CHUNK // GSPLIT

    def gather(j, rows, sems):
        for g in range(GSPLIT):
            pltpu.async_copy(x_hbm.at[src_v.at[j, pl.ds(g * H, H)]],
                             rows.at[pl.ds(g * H, H)], sems[g])

    def gwait(j, rows, sems):
        for g in range(GSPLIT):
            pltpu.make_async_copy(x_hbm.at[src_v.at[j, pl.ds(g * H, H)]],
                                  rows.at[pl.ds(g * H, H)], sems[g]).wait()

    bufs = ((rows_a, sems_a), (rows_b, sems_b), (rows_c, sems_c))

    def stage(st, carry):
        idx_wait(st)
        gather(0, rows_a, sems_a)
        gather(1, rows_b, sems_b)

        def step(j, r):
            rows, sems = bufs[r]
            gwait(j, rows, sems)

            @pl.when(j + 2 < IDXB)
            def _():
                nrows, nsems = bufs[(r + 2) % 3]
                gather(j + 2, nrows, nsems)

            pltpu.sync_copy(rows, acc.at[dst_v.at[j]], add=True)
            hist(j)

        def tri(t, c1):
            step(3 * t, 0)
            step(3 * t + 1, 1)
            step(3 * t + 2, 2)
            return c1

        lax.fori_loop(0, IDXB // 3, tri, 0)
        step(IDXB - 1, (IDXB - 1) % 3)

        @pl.when(st + 1 < CPW // IDXB)
        def _():
            idx_load(st + 1)

        return carry

    lax.fori_loop(0, CPW // IDXB, stage, 0)
    plsc.subcore_barrier()

    def wfire(i, carry):
        ck = sid + i * 16

        @pl.when(ck < ROW_CHUNKS)
        def _():
            pltpu.async_copy(
                acc.at[pl.ds(ck * CHUNK, CHUNK)],
                sum_hbm.at[pl.ds(cid * N_NODES + ck * CHUNK, CHUNK)],
                wsem,
            )

        return carry

    lax.fori_loop(0, (ROW_CHUNKS + 15) // 16, wfire, 0)
    pltpu.sync_copy(cnt_v, cnt_hbm.at[pl.ds(wid * N_NODES, N_NODES)])

    def wdrain(i, carry):
        ck = sid + i * 16

        @pl.when(ck < ROW_CHUNKS)
        def _():
            pltpu.make_async_copy(
                acc.at[pl.ds(ck * CHUNK, CHUNK)],
                sum_hbm.at[pl.ds(cid * N_NODES + ck * CHUNK, CHUNK)],
                wsem,
            ).wait()

        return carry

    lax.fori_loop(0, (ROW_CHUNKS + 15) // 16, wdrain, 0)


_sc_aggregate = pl.kernel(
    _sc_body,
    out_type=(
        jax.ShapeDtypeStruct((2 * N_NODES, D_IN), jnp.float32),
        jax.ShapeDtypeStruct((NW * N_NODES,), jnp.float32),
    ),
    mesh=plsc.VectorSubcoreMesh(core_axis_name="c", subcore_axis_name="s"),
    compiler_params=pltpu.CompilerParams(needs_layout_passes=False),
    scratch_types=[
        pltpu.VMEM_SHARED((N_NODES, D_IN), jnp.float32),
        pltpu.VMEM((IDXB, CHUNK), jnp.int32),
        pltpu.VMEM((IDXB, CHUNK), jnp.int32),
        pltpu.VMEM((CHUNK, D_IN), jnp.float32),
        pltpu.VMEM((CHUNK, D_IN), jnp.float32),
        pltpu.VMEM((CHUNK, D_IN), jnp.float32),
        pltpu.VMEM((N_NODES,), jnp.float32),
        [pltpu.SemaphoreType.DMA] * GSPLIT,
        [pltpu.SemaphoreType.DMA] * GSPLIT,
        [pltpu.SemaphoreType.DMA] * GSPLIT,
        pltpu.SemaphoreType.DMA,
        pltpu.SemaphoreType.DMA,
        pltpu.SemaphoreType.DMA,
    ],
)


def _tc_body(p_ref, c_ref, wg_ref, bg_ref, wf_ref, bf_ref, o_ref):
    p = p_ref[...]
    s = p[0] + p[1]
    cnt = jnp.maximum(jnp.sum(c_ref[...], axis=1), 1.0)[:, None]
    k = jax.lax.dot_general(wg_ref[...], wf_ref[...],
                            (((0,), (1,)), ((), ())),
                            preferred_element_type=jnp.float32)
    t = jnp.dot(s / cnt, k, preferred_element_type=jnp.float32)
    b = jax.lax.dot_general(bg_ref[...], wf_ref[...],
                            (((1,), (1,)), ((), ())),
                            preferred_element_type=jnp.float32)
    o_ref[...] = t + b + bf_ref[...]


def _tc_dense(partials, counts, wg, bg, wf, bf):
    R = 1000
    grid = (N_NODES // R,)
    return pl.pallas_call(
        _tc_body,
        grid=grid,
        in_specs=[
            pl.BlockSpec((2, R, D_IN), lambda i: (0, i, 0)),
            pl.BlockSpec((R, NW), lambda i: (i, 0)),
            pl.BlockSpec((D_IN, D_IN), lambda i: (0, 0)),
            pl.BlockSpec((1, D_IN), lambda i: (0, 0)),
            pl.BlockSpec((3, D_IN), lambda i: (0, 0)),
            pl.BlockSpec((1, 3), lambda i: (0, 0)),
        ],
        out_specs=pl.BlockSpec((R, 3), lambda i: (i, 0)),
        out_shape=jax.ShapeDtypeStruct((N_NODES, 3), jnp.float32),
    )(partials, counts, wg, bg, wf, bf)


@jax.jit
def kernel(x, edge_index, W_gnn, b_gnn, W_fc, b_fc):
    ei = edge_index.astype(jnp.int32).reshape(
        2, NW, CPW // IDXB, IDXB, CHUNK)
    zrow = jnp.zeros((CHUNK, D_IN), x.dtype)
    flat, cnt = _sc_aggregate(x, ei, zrow)
    partials = flat.reshape(2, N_NODES, D_IN)
    counts = cnt.reshape(NW, N_NODES).T
    return _tc_dense(partials, counts, W_gnn, b_gnn.reshape(1, D_IN),
                     W_fc, b_fc.reshape(1, 3))
